# Initial kernel scaffold; baseline (speedup 1.0000x reference)
import jax
import jax.numpy as jnp
from jax.experimental import pallas as pl


def kernel(x, edge_index_phys, edge_index_align, edge_index_logic, edge_attr_phys, edge_attr_logic, params):
    raise NotImplementedError("write your pallas kernel here")


# speedup vs baseline: 3.2088x; 3.2088x over previous
import functools
import jax
import jax.numpy as jnp
from jax import lax
from jax.experimental import pallas as pl
from jax.experimental.pallas import tpu as pltpu
from jax.experimental.pallas import tpu_sc as plsc

N = 10000
E = 160000
HID = 256
HEADS = 8
HD = 32

NC = 2
NS = 16
NW = NC * NS

CHUNK = 128
NCHUNKS = E // CHUNK
NPAD = 10240
ROWS_T = NPAD // NS
PWC = 128

MBLK = 400
EBLK = 640


def _mm_body(x_ref, w_ref, b_ref, o_ref, *, relu):
    r = jnp.dot(x_ref[...], w_ref[...], preferred_element_type=jnp.float32)
    r = r + b_ref[...]
    if relu:
        r = jnp.maximum(r, 0.0)
    o_ref[...] = r


def _mm(x, w, b, relu=False):
    m, k = x.shape
    n = w.shape[1]
    grid = (m // MBLK,)
    return pl.pallas_call(
        functools.partial(_mm_body, relu=relu),
        grid=grid,
        in_specs=[
            pl.BlockSpec((MBLK, k), lambda i: (i, 0)),
            pl.BlockSpec((k, n), lambda i: (0, 0)),
            pl.BlockSpec((1, n), lambda i: (0, 0)),
        ],
        out_specs=pl.BlockSpec((MBLK, n), lambda i: (i, 0)),
        out_shape=jax.ShapeDtypeStruct((m, n), jnp.float32),
    )(x, w, b)


def _proj_body(x_ref, w_ref, b_ref, o_ref, cm_ref):
    i = pl.program_id(0)
    r = jnp.dot(x_ref[...], w_ref[...], preferred_element_type=jnp.float32)
    r = r + b_ref[...]
    o_ref[...] = r
    pm = jnp.max(jnp.abs(r), axis=0, keepdims=True)
    pm = jnp.broadcast_to(pm, cm_ref.shape)

    @pl.when(i == 0)
    def _():
        cm_ref[...] = pm

    @pl.when(i > 0)
    def _():
        cm_ref[...] = jnp.maximum(cm_ref[...], pm)


def _proj(x, w, b):
    m, k = x.shape
    n = w.shape[1]
    grid = (m // MBLK,)
    return pl.pallas_call(
        _proj_body,
        grid=grid,
        in_specs=[
            pl.BlockSpec((MBLK, k), lambda i: (i, 0)),
            pl.BlockSpec((k, n), lambda i: (0, 0)),
            pl.BlockSpec((1, n), lambda i: (0, 0)),
        ],
        out_specs=[
            pl.BlockSpec((MBLK, n), lambda i: (i, 0)),
            pl.BlockSpec((8, n), lambda i: (0, 0)),
        ],
        out_shape=[
            jax.ShapeDtypeStruct((m, n), jnp.float32),
            jax.ShapeDtypeStruct((8, n), jnp.float32),
        ],
    )(x, w, b)


_MESH = plsc.VectorSubcoreMesh(core_axis_name="c", subcore_axis_name="s")

MAXCH_G = -(-NCHUNKS // NW)


@functools.partial(
    pl.kernel,
    out_type=[
        jax.ShapeDtypeStruct((E, HID), jnp.float32),
        jax.ShapeDtypeStruct((E, HID), jnp.float32),
    ],
    mesh=_MESH,
    scratch_types=[
        pltpu.VMEM((CHUNK,), jnp.int32),
        pltpu.VMEM((CHUNK, HID), jnp.float32),
        pltpu.SemaphoreType.DMA,
    ],
)
def _sc_gather(xl_hbm, xr_hbm, src_hbm, dst_hbm, gl_hbm, gr_hbm, idx_v, rows_v, sem):
    wid = lax.axis_index("s") * NC + lax.axis_index("c")

    def step(j, _):
        k = wid + j * NW

        @pl.when(k < NCHUNKS)
        def _():
            base = k * CHUNK
            pltpu.sync_copy(src_hbm.at[pl.ds(base, CHUNK)], idx_v)
            pltpu.async_copy(xl_hbm.at[idx_v], rows_v, sem).wait()
            pltpu.sync_copy(rows_v, gl_hbm.at[pl.ds(base, CHUNK)])
            pltpu.sync_copy(dst_hbm.at[pl.ds(base, CHUNK)], idx_v)
            pltpu.async_copy(xr_hbm.at[idx_v], rows_v, sem).wait()
            pltpu.sync_copy(rows_v, gr_hbm.at[pl.ds(base, CHUNK)])

        return ()

    lax.fori_loop(0, MAXCH_G, step, ())


MAXCH_S = -(-NCHUNKS // NS)


@functools.partial(
    pl.kernel,
    out_type=jax.ShapeDtypeStruct((NPAD, 2 * PWC), jnp.float32),
    mesh=_MESH,
    scratch_types=[
        pltpu.VMEM_SHARED((NPAD, PWC), jnp.float32),
        pltpu.VMEM((CHUNK,), jnp.int32),
        pltpu.VMEM((CHUNK, PWC), jnp.float32),
    ],
)
def _sc_scatter(payload_hbm, dst_hbm, zeros_hbm, out_hbm, acc, idx_v, pbuf):
    cid = lax.axis_index("c")
    sid = lax.axis_index("s")
    col0 = cid * PWC

    row0 = sid * ROWS_T
    pltpu.sync_copy(zeros_hbm.at[pl.ds(row0, ROWS_T)], acc.at[pl.ds(row0, ROWS_T)])
    plsc.subcore_barrier()

    def step(j, _):
        k = sid + j * NS

        @pl.when(k < NCHUNKS)
        def _():
            base = k * CHUNK
            pltpu.sync_copy(dst_hbm.at[pl.ds(base, CHUNK)], idx_v)
            pltpu.sync_copy(payload_hbm.at[pl.ds(base, CHUNK), pl.ds(col0, PWC)], pbuf)
            pltpu.sync_copy(pbuf, acc.at[idx_v], add=True)

        return ()

    lax.fori_loop(0, MAXCH_S, step, ())
    plsc.subcore_barrier()
    pltpu.sync_copy(acc.at[pl.ds(row0, ROWS_T)],
                    out_hbm.at[pl.ds(row0, ROWS_T), pl.ds(col0, PWC)])


def _edge_body(gl_ref, gr_ref, ea_ref, we_ref, a_ref, bb_ref, st_ref, ow_ref, os_ref):
    gl = gl_ref[...]
    z = gl + gr_ref[...]
    z = z + jnp.dot(ea_ref[...], we_ref[...], preferred_element_type=jnp.float32)
    e = jnp.where(z >= 0.0, z, 0.2 * z)
    logits = jnp.dot(e, a_ref[...], preferred_element_type=jnp.float32)
    e2 = jnp.exp(logits - st_ref[...])
    ow_ref[...] = gl * jnp.dot(e2, bb_ref[...], preferred_element_type=jnp.float32)
    pad = jnp.zeros((os_ref.shape[0], PWC - HEADS), jnp.float32)
    os_ref[...] = jnp.concatenate([e2, pad, e2, pad], axis=1)


def _edge(gl, gr, ea8, we8, amat, bmat, stab):
    grid = (E // EBLK,)
    return pl.pallas_call(
        _edge_body,
        grid=grid,
        in_specs=[
            pl.BlockSpec((EBLK, HID), lambda i: (i, 0)),
            pl.BlockSpec((EBLK, HID), lambda i: (i, 0)),
            pl.BlockSpec((EBLK, 8), lambda i: (i, 0)),
            pl.BlockSpec((8, HID), lambda i: (0, 0)),
            pl.BlockSpec((HID, 8), lambda i: (0, 0)),
            pl.BlockSpec((8, HID), lambda i: (0, 0)),
            pl.BlockSpec((1, 8), lambda i: (0, 0)),
        ],
        out_specs=[
            pl.BlockSpec((EBLK, HID), lambda i: (i, 0)),
            pl.BlockSpec((EBLK, 2 * PWC), lambda i: (i, 0)),
        ],
        out_shape=[
            jax.ShapeDtypeStruct((E, HID), jnp.float32),
            jax.ShapeDtypeStruct((E, 2 * PWC), jnp.float32),
        ],
    )(gl, gr, ea8, we8, amat, bmat, stab)


def _comb_body(wp_ref, sp_ref, wa_ref, sa_ref, wl_ref, sl_ref, bb_ref, bias_ref, o_ref):
    bb = bb_ref[...]
    tot = bias_ref[...]
    for w_ref, s_ref in ((wp_ref, sp_ref), (wa_ref, sa_ref), (wl_ref, sl_ref)):
        den = jnp.dot(s_ref[...][:, 0:HEADS], bb, preferred_element_type=jnp.float32)
        tot = tot + jnp.where(den > 0.0, w_ref[...] / den, 0.0)
    o_ref[...] = jnp.where(tot > 0.0, tot, jnp.exp(jnp.minimum(tot, 0.0)) - 1.0)


def _combine(accs, bmat, bias_sum):
    grid = (N // MBLK,)
    wspec = pl.BlockSpec((MBLK, HID), lambda i: (i, 0))
    sspec = pl.BlockSpec((MBLK, 2 * PWC), lambda i: (i, 0))
    return pl.pallas_call(
        _comb_body,
        grid=grid,
        in_specs=[
            wspec, sspec, wspec, sspec, wspec, sspec,
            pl.BlockSpec((8, HID), lambda i: (0, 0)),
            pl.BlockSpec((1, HID), lambda i: (0, 0)),
        ],
        out_specs=pl.BlockSpec((MBLK, HID), lambda i: (i, 0)),
        out_shape=jax.ShapeDtypeStruct((N, HID), jnp.float32),
    )(accs[0][0], accs[0][1], accs[1][0], accs[1][1], accs[2][0], accs[2][1],
      bmat, bias_sum)


def _heads_body(h_ref, w1_ref, b1_ref, w2_ref, b2_ref, act_ref, hs_ref):
    i = pl.program_id(0)
    h = h_ref[...]
    a1 = jnp.maximum(jnp.dot(h, w1_ref[...], preferred_element_type=jnp.float32)
                     + b1_ref[...], 0.0)
    act_ref[...] = jnp.dot(a1, w2_ref[...], preferred_element_type=jnp.float32) + b2_ref[...]
    ps = jnp.broadcast_to(jnp.sum(h, axis=0, keepdims=True), hs_ref.shape)

    @pl.when(i == 0)
    def _():
        hs_ref[...] = ps

    @pl.when(i > 0)
    def _():
        hs_ref[...] = hs_ref[...] + ps


def _heads(h, w1, b1, w2, b2):
    grid = (N // MBLK,)
    return pl.pallas_call(
        _heads_body,
        grid=grid,
        in_specs=[
            pl.BlockSpec((MBLK, HID), lambda i: (i, 0)),
            pl.BlockSpec((HID, HID), lambda i: (0, 0)),
            pl.BlockSpec((1, HID), lambda i: (0, 0)),
            pl.BlockSpec((HID, 8), lambda i: (0, 0)),
            pl.BlockSpec((1, 8), lambda i: (0, 0)),
        ],
        out_specs=[
            pl.BlockSpec((MBLK, 8), lambda i: (i, 0)),
            pl.BlockSpec((8, HID), lambda i: (0, 0)),
        ],
        out_shape=[
            jax.ShapeDtypeStruct((N, 8), jnp.float32),
            jax.ShapeDtypeStruct((8, HID), jnp.float32),
        ],
    )(h, w1, b1, w2, b2)


def _value_body(hs_ref, w1_ref, b1_ref, w2_ref, b2_ref, o_ref):
    g = hs_ref[0:1, :] * (1.0 / N)
    v1 = jnp.maximum(jnp.dot(g, w1_ref[...], preferred_element_type=jnp.float32)
                     + b1_ref[...], 0.0)
    v = jnp.dot(v1, w2_ref[...], preferred_element_type=jnp.float32) + b2_ref[...]
    o_ref[...] = jnp.broadcast_to(v, o_ref.shape)


def _value(hsum, w1, b1, w2, b2):
    return pl.pallas_call(
        _value_body,
        in_specs=[
            pl.BlockSpec((8, HID), lambda: (0, 0)),
            pl.BlockSpec((HID, HID), lambda: (0, 0)),
            pl.BlockSpec((1, HID), lambda: (0, 0)),
            pl.BlockSpec((HID, 8), lambda: (0, 0)),
            pl.BlockSpec((1, 8), lambda: (0, 0)),
        ],
        out_specs=pl.BlockSpec((8, 8), lambda: (0, 0)),
        out_shape=jax.ShapeDtypeStruct((8, 8), jnp.float32),
    )(hsum, w1, b1, w2, b2)


def kernel(x, edge_index_phys, edge_index_align, edge_index_logic,
           edge_attr_phys, edge_attr_logic, params):
    f32 = jnp.float32
    eye8 = jnp.eye(HEADS, dtype=f32)
    bmat = jnp.kron(eye8, jnp.ones((1, HD), f32))
    zeros_acc = jnp.zeros((NPAD, PWC), f32)

    etypes = [
        ("phys", edge_index_phys, edge_attr_phys),
        ("align", edge_index_align, None),
        ("logic", edge_index_logic, edge_attr_logic),
    ]
    eprep = []
    for name, ei, ea in etypes:
        src = ei[0]
        dst = ei[1]
        if ea is not None:
            ea8 = jnp.pad(ea, ((0, 0), (0, 7)))
            maxea = jnp.max(jnp.abs(ea))
        else:
            ea8 = jnp.zeros((E, 8), f32)
            maxea = jnp.zeros((), f32)
        eprep.append((name, src, dst, ea8, maxea))

    xp = jnp.pad(x, ((0, 0), (0, 4)))
    encW = jnp.pad(params["enc_W"], ((0, 4), (0, 0)))
    h = _mm(xp, encW, params["enc_b"].reshape(1, HID))

    for lp in params["layers"]:
        wcat = jnp.concatenate(
            [m for nm in ("phys", "align", "logic") for m in (lp[nm]["Wl"], lp[nm]["Wr"])],
            axis=1)
        bcat = jnp.concatenate(
            [v for nm in ("phys", "align", "logic") for v in (lp[nm]["bl"], lp[nm]["br"])]
        ).reshape(1, 6 * HID)
        proj, cmax = _proj(h, wcat, bcat)
        cm = cmax[0]

        accs = []
        for t, (name, src, dst, ea8, maxea) in enumerate(eprep):
            p = lp[name]
            offl, offr = 2 * t * HID, (2 * t + 1) * HID
            xl = lax.slice(proj, (0, offl), (N, offl + HID))
            xr = lax.slice(proj, (0, offr), (N, offr + HID))
            mx = cm[offl:offl + HID] + cm[offr:offr + HID]
            if name != "align":
                we_row = p["We"][0]
                mx = mx + maxea * jnp.abs(we_row)
                we8 = jnp.pad(we_row.reshape(1, HID), ((0, 7), (0, 0)))
            else:
                we8 = jnp.zeros((8, HID), f32)
            stab = jnp.sum(jnp.abs(p["att"]) * mx.reshape(HEADS, HD), axis=-1)
            amat = (p["att"].reshape(HEADS, HD, 1) * eye8[:, None, :]).reshape(HID, HEADS)

            gl, gr = _sc_gather(xl, xr, src, dst)
            pay_w, pay_s = _edge(gl, gr, ea8, we8, amat, bmat, stab.reshape(1, HEADS))
            out_w = _sc_scatter(pay_w, dst, zeros_acc)
            out_s = _sc_scatter(pay_s, dst, zeros_acc)
            accs.append((out_w, out_s))

        bias_sum = (lp["phys"]["bias"] + lp["align"]["bias"] + lp["logic"]["bias"]).reshape(1, HID)
        h = _combine(accs, bmat, bias_sum)

    aw2 = params["aW2"]
    action, hsum = _heads(h, params["aW1"], params["ab1"].reshape(1, HID),
                          aw2, params["ab2"].reshape(1, 8))
    vout = _value(hsum, params["cW1"], params["cb1"].reshape(1, HID),
                  jnp.pad(params["cW2"], ((0, 0), (0, 7))),
                  jnp.pad(params["cb2"], (0, 7)).reshape(1, 8))
    value = vout[0, 0:1]
    return (action, value)
